# baseline (speedup 1.0000x reference)
import functools

import jax
import jax.numpy as jnp
from jax import lax
from jax.experimental import pallas as pl
from jax.experimental.pallas import tpu as pltpu
from jax.experimental.pallas import tpu_sc as plsc

B = 16
A = 100000
G = 64
NC = 2
NS = 16
L = 16

_mesh = plsc.VectorSubcoreMesh(
    core_axis_name="c", subcore_axis_name="s", num_cores=NC, num_subcores=NS)


@functools.partial(
    pl.kernel,
    out_type=jax.ShapeDtypeStruct((B * G,), jnp.float32),
    mesh=_mesh,
    compiler_params=pltpu.CompilerParams(needs_layout_passes=False),
    scratch_types=[
        pltpu.VMEM((G,), jnp.int32),
        pltpu.VMEM((G,), jnp.float32),
        pltpu.VMEM((A,), jnp.int32),
    ],
)
def _sc_dedup(aidx_hbm, mask_out, idxim, mask_v, table):
    w = lax.axis_index("c") * NS + lax.axis_index("s")

    @pl.when(w < B)
    def _dedup():
        pltpu.sync_copy(aidx_hbm.at[pl.ds(w * G, G)], idxim)
        for j in range(G // L):
            ids = lax.iota(jnp.int32, L) + L * j
            chunk = idxim[pl.ds(L * j, L)]
            plsc.store_scatter(table, [chunk], ids)
        for j in range(G // L):
            ids = lax.iota(jnp.int32, L) + L * j
            chunk = idxim[pl.ds(L * j, L)]
            got = plsc.load_gather(table, [chunk])
            mask_v[pl.ds(L * j, L)] = jnp.where(got == ids, 1.0, 0.0)
        pltpu.sync_copy(mask_v, mask_out.at[pl.ds(w * G, G)])


def _tc_body(idx_smem, pc_blk, aidx_col, mask_col, gx1, gy1, gx2, gy2,
             pbt_any, anct_any,
             lc_ref, le_ref, conf_w, pb_w, anc_w, acc, sem):
    step = pl.program_id(0)
    rows = B // 2

    def win_of(i):
        a = idx_smem[i]
        return pl.multiple_of((a // 128) * 128, 128)

    def winloads(b, g, fire):
        i = b * G + g
        aw = win_of(i)
        if fire:
            pltpu.make_async_copy(
                pbt_any.at[b, :, pl.ds(aw, 128)],
                pb_w.at[:, i, :], sem).start()
        conf_w[pl.ds(i, 1), :] = pc_blk[b % rows:b % rows + 1, pl.ds(aw, 128)]
        anc_w[i] = anct_any[0, :, pl.ds(aw, 128)]

    @pl.when(step == 0)
    def _fire():
        for b in range(rows):
            lax.fori_loop(0, G, lambda g, c, b=b: (winloads(b, g, True), c)[1],
                          0, unroll=16)
        for b in range(rows, B):
            def fire1(g, carry, b=b):
                i = b * G + g
                aw = win_of(i)
                pltpu.make_async_copy(
                    pbt_any.at[b, :, pl.ds(aw, 128)],
                    pb_w.at[:, i, :], sem).start()
                return carry

            lax.fori_loop(0, G, fire1, 0, unroll=16)

    @pl.when(step == 1)
    def _winloads1():
        for b in range(rows, B):
            lax.fori_loop(0, G, lambda g, c, b=b: (winloads(b, g, False), c)[1],
                          0, unroll=16)

    x = pc_blk[...]
    conf = 1.0 / (1.0 + jnp.exp(-x))
    s = jnp.sum(jnp.maximum(jnp.log(1.0 - conf), -100.0))

    @pl.when(step == 0)
    def _init():
        acc[0] = s

    @pl.when(step == pl.num_programs(0) - 1)
    def _final():
        total = acc[0] + s
        pltpu.make_async_copy(pb_w, pb_w, sem).wait()

        lane = aidx_col[...] & 127
        iota128 = lax.broadcasted_iota(jnp.int32, (B * G, 128), 1)
        m = lane == iota128

        def sel(win2d):
            col = jnp.sum(jnp.where(m, win2d, 0.0), axis=1, keepdims=True)
            return col.reshape(8, 128)

        cg = sel(conf_w[...])
        pbx, pby = sel(pb_w[0]), sel(pb_w[1])
        pbw, pbh = sel(pb_w[2]), sel(pb_w[3])
        ax1, ay1 = sel(anc_w[:, 0, :]), sel(anc_w[:, 1, :])
        ax2, ay2 = sel(anc_w[:, 2, :]), sel(anc_w[:, 3, :])
        tx1, ty1 = gx1[...], gy1[...]
        tx2, ty2 = gx2[...], gy2[...]

        acx = (ax1 + ax2) * 0.5
        acy = (ay1 + ay2) * 0.5
        aw = ax2 - ax1
        ah = ay2 - ay1
        cx = acx + pbx * 0.1 * aw
        cy = acy + pby * 0.1 * ah
        bw = aw * jnp.exp(pbw * 0.2)
        bh = ah * jnp.exp(pbh * 0.2)
        px1 = cx - 0.5 * bw
        py1 = cy - 0.5 * bh
        px2 = cx + 0.5 * bw
        py2 = cy + 0.5 * bh

        ex1 = jnp.minimum(px1, tx1)
        ey1 = jnp.minimum(py1, ty1)
        ix1 = jnp.maximum(px1, tx1)
        iy1 = jnp.maximum(py1, ty1)
        ix2 = jnp.minimum(px2, tx2)
        iy2 = jnp.minimum(py2, ty2)
        xmin = jnp.minimum(ix1, ix2)
        ymin = jnp.minimum(iy1, iy2)
        xmax = jnp.maximum(ix1, ix2)
        ymax = jnp.maximum(iy1, iy2)
        inter = ((ix2 - ex1) * (iy2 - ey1) + (xmin - ex1) * (ymin - ey1)
                 - (ix1 - ex1) * (ymax - ey1) - (xmax - ex1) * (iy1 - ey1))
        union = ((px2 - px1) * (py2 - py1) + (tx2 - tx1) * (ty2 - ty1)
                 - inter + 1e-07)
        ious = 1.0 - inter / union
        ss = jnp.where(ious < 0.1, 1.0, 0.0)
        el = 0.5 * ss * ious * ious / 0.1 + (1.0 - ss) * (ious - 0.05)

        r8 = lax.broadcasted_iota(jnp.int32, (8, 128), 0)
        l8 = lax.broadcasted_iota(jnp.int32, (8, 128), 1)
        img = (r8 * 128 + l8) // G
        le = jnp.float32(0.0)
        for b in range(B):
            le = le + jnp.min(jnp.where(img == b, el, jnp.inf))
        le_ref[0, 0] = le

        cc = 1.0 / (1.0 + jnp.exp(-cg))
        lpos = jnp.maximum(jnp.log(cc), -100.0)
        lneg = jnp.maximum(jnp.log(1.0 - cc), -100.0)
        corr = jnp.sum((-lpos + 0.002 * lneg) * mask_col[...])
        lc_ref[0, 0] = -0.002 * total + corr


def _tc_main(pc2d, aidx_flat, aidx_col, mask_col, gcols, pbt, anct):
    nblk = 2
    return pl.pallas_call(
        _tc_body,
        grid=(nblk,),
        in_specs=[
            pl.BlockSpec(memory_space=pltpu.SMEM),
            pl.BlockSpec((B // nblk, A), lambda i: (i, 0)),
            pl.BlockSpec((B * G, 1), lambda i: (0, 0)),
            pl.BlockSpec((8, 128), lambda i: (0, 0)),
            pl.BlockSpec((8, 128), lambda i: (0, 0)),
            pl.BlockSpec((8, 128), lambda i: (0, 0)),
            pl.BlockSpec((8, 128), lambda i: (0, 0)),
            pl.BlockSpec((8, 128), lambda i: (0, 0)),
            pl.BlockSpec(memory_space=pltpu.MemorySpace.HBM),
            pl.BlockSpec((1, 4, A), lambda i: (0, 0, 0)),
        ],
        out_specs=[
            pl.BlockSpec(memory_space=pltpu.SMEM),
            pl.BlockSpec(memory_space=pltpu.SMEM),
        ],
        out_shape=[
            jax.ShapeDtypeStruct((1, 1), jnp.float32),
            jax.ShapeDtypeStruct((1, 1), jnp.float32),
        ],
        scratch_shapes=[
            pltpu.VMEM((B * G, 128), jnp.float32),
            pltpu.VMEM((4, B * G, 128), jnp.float32),
            pltpu.VMEM((B * G, 4, 128), jnp.float32),
            pltpu.SMEM((1,), jnp.float32),
            pltpu.SemaphoreType.DMA,
        ],
    )(aidx_flat, pc2d, aidx_col, mask_col, *gcols, pbt, anct)


def kernel(pred_conf, pred_boxes, boxes, anchor_indexes, cls, anchors):
    aidx_flat = anchor_indexes.reshape(-1).astype(jnp.int32)
    mask = _sc_dedup(aidx_flat)

    pbt = jnp.transpose(pred_boxes, (0, 2, 1))
    anct = jnp.transpose(anchors.reshape(1, A, 4), (0, 2, 1))
    gt2 = boxes.reshape(-1, 4)
    gcols = [gt2[:, c].reshape(8, 128) for c in range(4)]

    lc, le = _tc_main(pred_conf, aidx_flat, aidx_flat.reshape(-1, 1),
                      mask.reshape(8, 128), gcols, pbt, anct)
    return (lc.reshape(()), le.reshape(1))


# speedup vs baseline: 1.0141x; 1.0141x over previous
import functools

import jax
import jax.numpy as jnp
from jax import lax
from jax.experimental import pallas as pl
from jax.experimental.pallas import tpu as pltpu
from jax.experimental.pallas import tpu_sc as plsc

B = 16
A = 100000
G = 64
NC = 2
NS = 16
L = 16

_mesh = plsc.VectorSubcoreMesh(
    core_axis_name="c", subcore_axis_name="s", num_cores=NC, num_subcores=NS)


@functools.partial(
    pl.kernel,
    out_type=jax.ShapeDtypeStruct((B * G,), jnp.float32),
    mesh=_mesh,
    compiler_params=pltpu.CompilerParams(needs_layout_passes=False),
    scratch_types=[
        pltpu.VMEM((G,), jnp.int32),
        pltpu.VMEM((G,), jnp.float32),
        pltpu.VMEM((A,), jnp.int32),
    ],
)
def _sc_dedup(aidx_hbm, mask_out, idxim, mask_v, table):
    w = lax.axis_index("c") * NS + lax.axis_index("s")

    @pl.when(w < B)
    def _dedup():
        pltpu.sync_copy(aidx_hbm.at[pl.ds(w * G, G)], idxim)
        for j in range(G // L):
            ids = lax.iota(jnp.int32, L) + L * j
            chunk = idxim[pl.ds(L * j, L)]
            plsc.store_scatter(table, [chunk], ids)
        for j in range(G // L):
            ids = lax.iota(jnp.int32, L) + L * j
            chunk = idxim[pl.ds(L * j, L)]
            got = plsc.load_gather(table, [chunk])
            mask_v[pl.ds(L * j, L)] = jnp.where(got == ids, 1.0, 0.0)
        pltpu.sync_copy(mask_v, mask_out.at[pl.ds(w * G, G)])


def _tc_body(idx_smem, pc_blk, aidx_col, mask_col, gx1, gy1, gx2, gy2,
             pbt_any, anct_any,
             lc_ref, le_ref, conf_w, pb_w, aw0, aw1, aw2, aw3, acc, sem):
    step = pl.program_id(0)
    rows = B // 2
    anc_ws = (aw0, aw1, aw2, aw3)

    def win_of(i):
        a = idx_smem[i]
        return pl.multiple_of((a // 128) * 128, 128)

    def winloads(b, g, fire):
        i = b * G + g
        aw = win_of(i)
        if fire:
            pltpu.make_async_copy(
                pbt_any.at[b, :, pl.ds(aw, 128)],
                pb_w.at[:, i, :], sem).start()
        conf_w[pl.ds(i, 1), :] = pc_blk[b % rows:b % rows + 1, pl.ds(aw, 128)]
        awin = anct_any[0, :, pl.ds(aw, 128)]
        for c in range(4):
            anc_ws[c][pl.ds(i, 1), :] = awin[c:c + 1, :]

    @pl.when(step == 0)
    def _fire():
        for b in range(rows):
            lax.fori_loop(0, G, lambda g, c, b=b: (winloads(b, g, True), c)[1],
                          0, unroll=16)
        for b in range(rows, B):
            def fire1(g, carry, b=b):
                i = b * G + g
                aw = win_of(i)
                pltpu.make_async_copy(
                    pbt_any.at[b, :, pl.ds(aw, 128)],
                    pb_w.at[:, i, :], sem).start()
                return carry

            lax.fori_loop(0, G, fire1, 0, unroll=16)

    @pl.when(step == 1)
    def _winloads1():
        for b in range(rows, B):
            lax.fori_loop(0, G, lambda g, c, b=b: (winloads(b, g, False), c)[1],
                          0, unroll=16)

    x = pc_blk[...]
    conf = 1.0 / (1.0 + jnp.exp(-x))
    s = jnp.sum(jnp.maximum(jnp.log(1.0 - conf), -100.0))

    @pl.when(step == 0)
    def _init():
        acc[0] = s

    @pl.when(step == pl.num_programs(0) - 1)
    def _final():
        total = acc[0] + s
        pltpu.make_async_copy(pb_w, pb_w, sem).wait()

        lane = aidx_col[...] & 127
        iota128 = lax.broadcasted_iota(jnp.int32, (B * G, 128), 1)
        m = lane == iota128

        def sel(win2d):
            col = jnp.sum(jnp.where(m, win2d, 0.0), axis=1, keepdims=True)
            return col.reshape(8, 128)

        cg = sel(conf_w[...])
        pbx, pby = sel(pb_w[0]), sel(pb_w[1])
        pbw, pbh = sel(pb_w[2]), sel(pb_w[3])
        ax1, ay1 = sel(aw0[...]), sel(aw1[...])
        ax2, ay2 = sel(aw2[...]), sel(aw3[...])
        tx1, ty1 = gx1[...], gy1[...]
        tx2, ty2 = gx2[...], gy2[...]

        acx = (ax1 + ax2) * 0.5
        acy = (ay1 + ay2) * 0.5
        aw = ax2 - ax1
        ah = ay2 - ay1
        cx = acx + pbx * 0.1 * aw
        cy = acy + pby * 0.1 * ah
        bw = aw * jnp.exp(pbw * 0.2)
        bh = ah * jnp.exp(pbh * 0.2)
        px1 = cx - 0.5 * bw
        py1 = cy - 0.5 * bh
        px2 = cx + 0.5 * bw
        py2 = cy + 0.5 * bh

        ex1 = jnp.minimum(px1, tx1)
        ey1 = jnp.minimum(py1, ty1)
        ix1 = jnp.maximum(px1, tx1)
        iy1 = jnp.maximum(py1, ty1)
        ix2 = jnp.minimum(px2, tx2)
        iy2 = jnp.minimum(py2, ty2)
        xmin = jnp.minimum(ix1, ix2)
        ymin = jnp.minimum(iy1, iy2)
        xmax = jnp.maximum(ix1, ix2)
        ymax = jnp.maximum(iy1, iy2)
        inter = ((ix2 - ex1) * (iy2 - ey1) + (xmin - ex1) * (ymin - ey1)
                 - (ix1 - ex1) * (ymax - ey1) - (xmax - ex1) * (iy1 - ey1))
        union = ((px2 - px1) * (py2 - py1) + (tx2 - tx1) * (ty2 - ty1)
                 - inter + 1e-07)
        ious = 1.0 - inter / union
        ss = jnp.where(ious < 0.1, 1.0, 0.0)
        el = 0.5 * ss * ious * ious / 0.1 + (1.0 - ss) * (ious - 0.05)

        r8 = lax.broadcasted_iota(jnp.int32, (8, 128), 0)
        l8 = lax.broadcasted_iota(jnp.int32, (8, 128), 1)
        img = (r8 * 128 + l8) // G
        le = jnp.float32(0.0)
        for b in range(B):
            le = le + jnp.min(jnp.where(img == b, el, jnp.inf))
        le_ref[0, 0] = le

        cc = 1.0 / (1.0 + jnp.exp(-cg))
        lpos = jnp.maximum(jnp.log(cc), -100.0)
        lneg = jnp.maximum(jnp.log(1.0 - cc), -100.0)
        corr = jnp.sum((-lpos + 0.002 * lneg) * mask_col[...])
        lc_ref[0, 0] = -0.002 * total + corr


def _tc_main(pc2d, aidx_flat, aidx_col, mask_col, gcols, pbt, anct):
    nblk = 2
    return pl.pallas_call(
        _tc_body,
        grid=(nblk,),
        in_specs=[
            pl.BlockSpec(memory_space=pltpu.SMEM),
            pl.BlockSpec((B // nblk, A), lambda i: (i, 0)),
            pl.BlockSpec((B * G, 1), lambda i: (0, 0)),
            pl.BlockSpec((8, 128), lambda i: (0, 0)),
            pl.BlockSpec((8, 128), lambda i: (0, 0)),
            pl.BlockSpec((8, 128), lambda i: (0, 0)),
            pl.BlockSpec((8, 128), lambda i: (0, 0)),
            pl.BlockSpec((8, 128), lambda i: (0, 0)),
            pl.BlockSpec(memory_space=pltpu.MemorySpace.HBM),
            pl.BlockSpec((1, 4, A), lambda i: (0, 0, 0)),
        ],
        out_specs=[
            pl.BlockSpec(memory_space=pltpu.SMEM),
            pl.BlockSpec(memory_space=pltpu.SMEM),
        ],
        out_shape=[
            jax.ShapeDtypeStruct((1, 1), jnp.float32),
            jax.ShapeDtypeStruct((1, 1), jnp.float32),
        ],
        scratch_shapes=[
            pltpu.VMEM((B * G, 128), jnp.float32),
            pltpu.VMEM((4, B * G, 128), jnp.float32),
            pltpu.VMEM((B * G, 128), jnp.float32),
            pltpu.VMEM((B * G, 128), jnp.float32),
            pltpu.VMEM((B * G, 128), jnp.float32),
            pltpu.VMEM((B * G, 128), jnp.float32),
            pltpu.SMEM((1,), jnp.float32),
            pltpu.SemaphoreType.DMA,
        ],
    )(aidx_flat, pc2d, aidx_col, mask_col, *gcols, pbt, anct)


def kernel(pred_conf, pred_boxes, boxes, anchor_indexes, cls, anchors):
    aidx_flat = anchor_indexes.reshape(-1).astype(jnp.int32)
    mask = _sc_dedup(aidx_flat)

    pbt = jnp.transpose(pred_boxes, (0, 2, 1))
    anct = jnp.transpose(anchors.reshape(1, A, 4), (0, 2, 1))
    gt2 = boxes.reshape(-1, 4)
    gcols = [gt2[:, c].reshape(8, 128) for c in range(4)]

    lc, le = _tc_main(pred_conf, aidx_flat, aidx_flat.reshape(-1, 1),
                      mask.reshape(8, 128), gcols, pbt, anct)
    return (lc.reshape(()), le.reshape(1))
